# initial kernel scaffold (unmeasured)
import jax
import jax.numpy as jnp
from jax import lax
from jax.experimental import pallas as pl
from jax.experimental.pallas import tpu as pltpu

N_DEV = 4
SQ = 512
SKV = 2048
D = 1024
HQ = 8
DH = 128
SCALE = 0.08838834764831843


def _body(x_ref, wq_ref, wo_ref, k_ref, v_ref, out_ref,
          xg_ref, part_ref, ag_comm, rs_comm,
          ag_send, ag_recv, rs_send, rs_recv):
    j = lax.axis_index("i")
    left = (j + N_DEV - 1) % N_DEV
    right = (j + 1) % N_DEV

    barrier = pltpu.get_barrier_semaphore()
    for nbr in (left, right):
        pl.semaphore_signal(barrier, inc=1, device_id=(nbr,),
                            device_id_type=pl.DeviceIdType.MESH)
    pl.semaphore_wait(barrier, 2)

    xg_ref[pl.ds(j * SQ, SQ), :] = x_ref[...]
    ag_comm[0, :, :] = x_ref[...]
    for h in range(N_DEV - 1):
        s_slot, r_slot = h % 2, (h + 1) % 2
        rdma = pltpu.make_async_remote_copy(
            src_ref=ag_comm.at[s_slot], dst_ref=ag_comm.at[r_slot],
            send_sem=ag_send.at[s_slot], recv_sem=ag_recv.at[r_slot],
            device_id=(right,), device_id_type=pl.DeviceIdType.MESH)
        rdma.start()
        rdma.wait()
        origin = (j + (N_DEV - 1 - h)) % N_DEV
        xg_ref[pl.ds(origin * SQ, SQ), :] = ag_comm[r_slot]

    for c in range(N_DEV):
        xc = xg_ref[c * SQ:(c + 1) * SQ, :]
        q = jnp.dot(xc, wq_ref[...],
                    preferred_element_type=jnp.float32).astype(jnp.bfloat16)
        cols = []
        for hh in range(HQ):
            qh = q[:, hh * DH:(hh + 1) * DH]
            s = lax.dot_general(qh, k_ref[hh], (((1,), (1,)), ((), ())),
                                preferred_element_type=jnp.float32) * SCALE
            m = jnp.max(s, axis=1, keepdims=True)
            e = jnp.exp(s - m)
            l = jnp.sum(e, axis=1, keepdims=True)
            oh = lax.dot_general(e.astype(jnp.bfloat16), v_ref[hh],
                                 (((1,), (0,)), ((), ())),
                                 preferred_element_type=jnp.float32)
            cols.append((oh / l).astype(jnp.bfloat16))
        attn = jnp.concatenate(cols, axis=1)
        part_ref[c * SQ:(c + 1) * SQ, :] = jnp.dot(
            attn, wo_ref[...], preferred_element_type=jnp.float32)

    first = (j + N_DEV - 1) % N_DEV
    rs_comm[0, :, :] = part_ref[pl.ds(first * SQ, SQ), :]
    for s in range(N_DEV - 1):
        s_slot, r_slot = s % 2, (s + 1) % 2
        rdma = pltpu.make_async_remote_copy(
            src_ref=rs_comm.at[s_slot], dst_ref=rs_comm.at[r_slot],
            send_sem=rs_send.at[s_slot], recv_sem=rs_recv.at[r_slot],
            device_id=(right,), device_id_type=pl.DeviceIdType.MESH)
        rdma.start()
        rdma.wait()
        rc = (j + N_DEV - 2 - s) % N_DEV
        rs_comm[r_slot, :, :] = rs_comm[r_slot] + part_ref[pl.ds(rc * SQ, SQ), :]
    out_ref[...] = rs_comm[(N_DEV - 1) % 2]


def kernel(x, Wq, Wo, K_ext, V_ext):
    j = lax.axis_index("i")
    xb = x[0].astype(jnp.bfloat16)
    wq = Wq.astype(jnp.bfloat16)
    wo = Wo.astype(jnp.bfloat16)
    k = lax.dynamic_slice_in_dim(K_ext[0], j * HQ, HQ, axis=1)
    v = lax.dynamic_slice_in_dim(V_ext[0], j * HQ, HQ, axis=1)
    kb = jnp.transpose(k, (1, 0, 2)).astype(jnp.bfloat16)
    vb = jnp.transpose(v, (1, 0, 2)).astype(jnp.bfloat16)

    out = pl.pallas_call(
        _body,
        out_shape=jax.ShapeDtypeStruct((SQ, D), jnp.float32),
        in_specs=[pl.BlockSpec(memory_space=pltpu.VMEM)] * 5,
        out_specs=pl.BlockSpec(memory_space=pltpu.VMEM),
        scratch_shapes=[
            pltpu.VMEM((N_DEV * SQ, D), jnp.bfloat16),
            pltpu.VMEM((N_DEV * SQ, D), jnp.float32),
            pltpu.VMEM((2, SQ, D), jnp.bfloat16),
            pltpu.VMEM((2, SQ, D), jnp.float32),
            pltpu.SemaphoreType.DMA((2,)),
            pltpu.SemaphoreType.DMA((2,)),
            pltpu.SemaphoreType.DMA((2,)),
            pltpu.SemaphoreType.DMA((2,)),
        ],
        compiler_params=pltpu.CompilerParams(collective_id=0),
    )(xb, wq, wo, kb, vb)
    return out.reshape(1, SQ, D)


# baseline (device time: 207545 ns/iter reference)
import jax
import jax.numpy as jnp
from jax import lax
from jax.experimental import pallas as pl
from jax.experimental.pallas import tpu as pltpu

N_DEV = 4
SQ = 512
SKV = 2048
D = 1024
HQ = 8
DH = 128
SCALE = 0.08838834764831843


def _body(x_ref, wq_ref, wo_ref, k_ref, v_ref, out_ref,
          xg_ref, part_ref, ag_comm, rs_comm,
          ag_send, ag_recv, rs_send, rs_recv):
    j = lax.axis_index("i")
    left = (j + N_DEV - 1) % N_DEV
    right = (j + 1) % N_DEV

    barrier = pltpu.get_barrier_semaphore()
    for nbr in (left, right):
        pl.semaphore_signal(barrier, inc=1, device_id=(nbr,),
                            device_id_type=pl.DeviceIdType.MESH)
    pl.semaphore_wait(barrier, 2)

    xg_ref[pl.ds(j * SQ, SQ), :] = x_ref[...]
    ag_comm[0, :, :] = x_ref[...]
    for h in range(N_DEV - 1):
        s_slot, r_slot = h % 2, (h + 1) % 2
        rdma = pltpu.make_async_remote_copy(
            src_ref=ag_comm.at[s_slot], dst_ref=ag_comm.at[r_slot],
            send_sem=ag_send.at[s_slot], recv_sem=ag_recv.at[r_slot],
            device_id=(right,), device_id_type=pl.DeviceIdType.MESH)
        rdma.start()
        rdma.wait()
        origin = (j + (N_DEV - 1 - h)) % N_DEV
        xg_ref[pl.ds(origin * SQ, SQ), :] = ag_comm[r_slot]

    for c in range(N_DEV):
        xc = xg_ref[c * SQ:(c + 1) * SQ, :]
        q = jnp.dot(xc, wq_ref[...],
                    preferred_element_type=jnp.float32).astype(jnp.bfloat16)
        cols = []
        for hh in range(HQ):
            qh = q[:, hh * DH:(hh + 1) * DH]
            s = lax.dot_general(qh, k_ref[hh], (((1,), (1,)), ((), ())),
                                preferred_element_type=jnp.float32) * SCALE
            m = jnp.max(s, axis=1, keepdims=True)
            e = jnp.exp(s - m)
            l = jnp.sum(e, axis=1, keepdims=True)
            oh = lax.dot_general(e.astype(jnp.bfloat16), v_ref[hh],
                                 (((1,), (0,)), ((), ())),
                                 preferred_element_type=jnp.float32)
            cols.append((oh / l).astype(jnp.bfloat16))
        attn = jnp.concatenate(cols, axis=1)
        part_ref[c * SQ:(c + 1) * SQ, :] = jnp.dot(
            attn, wo_ref[...], preferred_element_type=jnp.float32)

    first = (j + N_DEV - 1) % N_DEV
    rs_comm[0, :, :] = part_ref[pl.ds(first * SQ, SQ), :]
    for s in range(N_DEV - 1):
        s_slot, r_slot = s % 2, (s + 1) % 2
        rdma = pltpu.make_async_remote_copy(
            src_ref=rs_comm.at[s_slot], dst_ref=rs_comm.at[r_slot],
            send_sem=rs_send.at[s_slot], recv_sem=rs_recv.at[r_slot],
            device_id=(right,), device_id_type=pl.DeviceIdType.MESH)
        rdma.start()
        rdma.wait()
        rc = (j + N_DEV - 2 - s) % N_DEV
        rs_comm[r_slot, :, :] = rs_comm[r_slot] + part_ref[pl.ds(rc * SQ, SQ), :]
    out_ref[...] = rs_comm[(N_DEV - 1) % 2]


def kernel(x, Wq, Wo, K_ext, V_ext):
    j = lax.axis_index("i")
    xb = x[0].astype(jnp.bfloat16)
    wq = Wq.astype(jnp.bfloat16)
    wo = Wo.astype(jnp.bfloat16)
    k = lax.dynamic_slice_in_dim(K_ext[0], j * HQ, HQ, axis=1)
    v = lax.dynamic_slice_in_dim(V_ext[0], j * HQ, HQ, axis=1)
    kb = jnp.transpose(k, (1, 0, 2)).astype(jnp.bfloat16)
    vb = jnp.transpose(v, (1, 0, 2)).astype(jnp.bfloat16)

    out = pl.pallas_call(
        _body,
        out_shape=jax.ShapeDtypeStruct((SQ, D), jnp.float32),
        in_specs=[pl.BlockSpec(memory_space=pltpu.VMEM)] * 5,
        out_specs=pl.BlockSpec(memory_space=pltpu.VMEM),
        scratch_shapes=[
            pltpu.VMEM((N_DEV * SQ, D), jnp.bfloat16),
            pltpu.VMEM((N_DEV * SQ, D), jnp.float32),
            pltpu.VMEM((2, SQ, D), jnp.bfloat16),
            pltpu.VMEM((2, SQ, D), jnp.float32),
            pltpu.SemaphoreType.DMA((2,)),
            pltpu.SemaphoreType.DMA((2,)),
            pltpu.SemaphoreType.DMA((2,)),
            pltpu.SemaphoreType.DMA((2,)),
        ],
        compiler_params=pltpu.CompilerParams(
            collective_id=0, vmem_limit_bytes=100 * 1024 * 1024),
    )(xb, wq, wo, kb, vb)
    return out.reshape(1, SQ, D)


# device time: 119576 ns/iter; 1.7357x vs baseline; 1.7357x over previous
import jax
import jax.numpy as jnp
from jax import lax
from jax.experimental import pallas as pl
from jax.experimental.pallas import tpu as pltpu

N_DEV = 4
SQ = 512
SKV = 2048
D = 1024
HQ = 8
DH = 128
SCALE = 0.08838834764831843


def _attn_partial(xc, wq_ref, wo_ref, k_ref, v_ref):
    q = jnp.dot(xc, wq_ref[...],
                preferred_element_type=jnp.float32).astype(jnp.bfloat16)
    cols = []
    for hh in range(HQ):
        qh = q[:, hh * DH:(hh + 1) * DH]
        s = lax.dot_general(qh, k_ref[hh], (((1,), (1,)), ((), ())),
                            preferred_element_type=jnp.float32) * SCALE
        m = jnp.max(s, axis=1, keepdims=True)
        e = jnp.exp(s - m)
        l = jnp.sum(e, axis=1, keepdims=True)
        oh = lax.dot_general(e.astype(jnp.bfloat16), v_ref[hh],
                             (((1,), (0,)), ((), ())),
                             preferred_element_type=jnp.float32)
        cols.append((oh / l).astype(jnp.bfloat16))
    attn = jnp.concatenate(cols, axis=1)
    return jnp.dot(attn, wo_ref[...], preferred_element_type=jnp.float32)


def _body(x_ref, wq_ref, wo_ref, k_ref, v_ref, out_ref,
          ag_buf, rs_send_buf, rs_recv_buf,
          ag_send, ag_recv, rs_send, rs_recv):
    j = lax.axis_index("i")
    left = (j + N_DEV - 1) % N_DEV
    right = (j + 1) % N_DEV

    barrier = pltpu.get_barrier_semaphore()
    for nbr in (left, right):
        pl.semaphore_signal(barrier, inc=1, device_id=(nbr,),
                            device_id_type=pl.DeviceIdType.MESH)
    pl.semaphore_wait(barrier, 2)

    ag_buf[0, :, :] = x_ref[...]
    ag = [
        pltpu.make_async_remote_copy(
            src_ref=ag_buf.at[h], dst_ref=ag_buf.at[h + 1],
            send_sem=ag_send.at[h], recv_sem=ag_recv.at[h],
            device_id=(right,), device_id_type=pl.DeviceIdType.MESH)
        for h in range(N_DEV - 1)
    ]
    rs = [
        pltpu.make_async_remote_copy(
            src_ref=rs_send_buf.at[s], dst_ref=rs_recv_buf.at[s],
            send_sem=rs_send.at[s], recv_sem=rs_recv.at[s],
            device_id=(right,), device_id_type=pl.DeviceIdType.MESH)
        for s in range(N_DEV - 1)
    ]

    ag[0].start()
    out_ref[...] = _attn_partial(x_ref[...], wq_ref, wo_ref, k_ref, v_ref)

    ag[0].wait_recv()
    ag[1].start()
    p = _attn_partial(ag_buf[1], wq_ref, wo_ref, k_ref, v_ref)
    rs_send_buf[0, :, :] = p.astype(jnp.bfloat16)
    rs[0].start()

    ag[1].wait_recv()
    ag[2].start()
    p = _attn_partial(ag_buf[2], wq_ref, wo_ref, k_ref, v_ref)
    rs[0].wait_recv()
    rs_send_buf[1, :, :] = (
        p + rs_recv_buf[0].astype(jnp.float32)).astype(jnp.bfloat16)
    rs[1].start()

    ag[2].wait_recv()
    p = _attn_partial(ag_buf[3], wq_ref, wo_ref, k_ref, v_ref)
    rs[1].wait_recv()
    rs_send_buf[2, :, :] = (
        p + rs_recv_buf[1].astype(jnp.float32)).astype(jnp.bfloat16)
    rs[2].start()

    rs[2].wait_recv()
    out_ref[...] = out_ref[...] + rs_recv_buf[2].astype(jnp.float32)

    for r in ag + rs:
        r.wait_send()


def kernel(x, Wq, Wo, K_ext, V_ext):
    j = lax.axis_index("i")
    xb = x[0].astype(jnp.bfloat16)
    wq = Wq.astype(jnp.bfloat16)
    wo = Wo.astype(jnp.bfloat16)
    k = lax.dynamic_slice_in_dim(K_ext[0], j * HQ, HQ, axis=1)
    v = lax.dynamic_slice_in_dim(V_ext[0], j * HQ, HQ, axis=1)
    kb = jnp.transpose(k, (1, 0, 2)).astype(jnp.bfloat16)
    vb = jnp.transpose(v, (1, 0, 2)).astype(jnp.bfloat16)

    out = pl.pallas_call(
        _body,
        out_shape=jax.ShapeDtypeStruct((SQ, D), jnp.float32),
        in_specs=[pl.BlockSpec(memory_space=pltpu.VMEM)] * 5,
        out_specs=pl.BlockSpec(memory_space=pltpu.VMEM),
        scratch_shapes=[
            pltpu.VMEM((N_DEV, SQ, D), jnp.bfloat16),
            pltpu.VMEM((N_DEV - 1, SQ, D), jnp.bfloat16),
            pltpu.VMEM((N_DEV - 1, SQ, D), jnp.bfloat16),
            pltpu.SemaphoreType.DMA((N_DEV - 1,)),
            pltpu.SemaphoreType.DMA((N_DEV - 1,)),
            pltpu.SemaphoreType.DMA((N_DEV - 1,)),
            pltpu.SemaphoreType.DMA((N_DEV - 1,)),
        ],
        compiler_params=pltpu.CompilerParams(
            collective_id=0, vmem_limit_bytes=100 * 1024 * 1024),
    )(xb, wq, wo, kb, vb)
    return out.reshape(1, SQ, D)
